# baseline (speedup 1.0000x reference)
import jax
import jax.numpy as jnp
from jax.experimental import pallas as pl
from jax.experimental.pallas import tpu as pltpu

H, D, K = 4096, 256, 8192
BETA = 0.25
N = 2 * 2048
BM = 256
NBLK = N // BM
KC = 2048
NKC = K // KC


def _vq_fused_kernel(emb_ref, wpre_ref, bpre_ref, cb_ref, e2_ref,
                     wpost_ref, bpost_ref,
                     out_ref, idx_ref, part_ref):
    z = jax.lax.dot_general(
        emb_ref[...], wpre_ref[...],
        dimension_numbers=(((1,), (1,)), ((), ())),
        preferred_element_type=jnp.float32)
    z = z + bpre_ref[...]
    x2 = jnp.sum(z ** 2, axis=1, keepdims=True)
    z2 = z + z

    fiota = jax.lax.broadcasted_iota(jnp.int32, (BM, KC), 1).astype(jnp.float32)
    mvec = None
    ckvec = None
    for c in range(NKC):
        xe2 = jax.lax.dot_general(
            z2, cb_ref[pl.ds(c * KC, KC), :],
            dimension_numbers=(((1,), (1,)), ((), ())),
            preferred_element_type=jnp.float32)
        dist = (x2 + e2_ref[:, pl.ds(c * KC, KC)]) - xe2
        if c == 0:
            mvec = dist
            ckvec = jnp.zeros((BM, KC), jnp.float32)
        else:
            upd = dist < mvec
            mvec = jnp.where(upd, dist, mvec)
            ckvec = jnp.where(upd, jnp.float32(c * KC), ckvec)

    m = jnp.min(mvec, axis=1, keepdims=True)
    fidx = ckvec + fiota
    idxf = jnp.min(jnp.where(mvec == m, fidx, jnp.float32(K)),
                   axis=1, keepdims=True)
    idx_ref[0, :, :] = idxf.astype(jnp.int32)

    x_q = jnp.zeros((BM, D), jnp.float32)
    for c in range(NKC):
        onehot = jnp.where(fiota == idxf - jnp.float32(c * KC),
                           jnp.float32(1.0), jnp.float32(0.0))
        x_q = x_q + jax.lax.dot_general(
            onehot, cb_ref[pl.ds(c * KC, KC), :],
            dimension_numbers=(((1,), (0,)), ((), ())),
            preferred_element_type=jnp.float32)

    x_q_st = z + (x_q - z)
    out = jax.lax.dot_general(
        x_q_st, wpost_ref[...],
        dimension_numbers=(((1,), (1,)), ((), ())),
        preferred_element_type=jnp.float32)
    out_ref[...] = out + bpost_ref[...]

    sum_sq = jnp.sum(m)
    lane = jax.lax.broadcasted_iota(jnp.int32, (1, 128), 1)
    part_ref[0, ...] = jnp.where(lane == 0, sum_sq, 0.0)


def kernel(embed, W_pre, b_pre, codebook, W_post, b_post, prior_logits):
    emb2d = embed.reshape(N, H)
    e2 = jnp.sum(codebook ** 2, axis=1)[None, :]

    embed_hat2d, idx3, parts = pl.pallas_call(
        _vq_fused_kernel,
        grid=(NBLK,),
        compiler_params=pltpu.CompilerParams(
            dimension_semantics=("parallel",)),
        in_specs=[
            pl.BlockSpec((BM, H), lambda i: (i, 0)),
            pl.BlockSpec((D, H), lambda i: (0, 0)),
            pl.BlockSpec((1, D), lambda i: (0, 0)),
            pl.BlockSpec((K, D), lambda i: (0, 0)),
            pl.BlockSpec((1, K), lambda i: (0, 0)),
            pl.BlockSpec((H, D), lambda i: (0, 0)),
            pl.BlockSpec((1, H), lambda i: (0, 0)),
        ],
        out_specs=[
            pl.BlockSpec((BM, H), lambda i: (i, 0)),
            pl.BlockSpec((1, BM, 1), lambda i: (i, 0, 0)),
            pl.BlockSpec((1, 1, 128), lambda i: (i, 0, 0)),
        ],
        out_shape=[
            jax.ShapeDtypeStruct((N, H), jnp.float32),
            jax.ShapeDtypeStruct((NBLK, BM, 1), jnp.int32),
            jax.ShapeDtypeStruct((NBLK, 1, 128), jnp.float32),
        ],
    )(emb2d, W_pre, b_pre.reshape(1, D), codebook, e2,
      W_post, b_post.reshape(1, H))

    embed_hat = embed_hat2d.reshape(embed.shape)
    idx = idx3.reshape(N)
    mean_sq = jnp.sum(parts[:, 0, 0]) / (N * D)
    vq_loss = mean_sq + BETA * mean_sq
    lse = jax.nn.logsumexp(prior_logits)
    sum_plog = jnp.sum(jnp.take(prior_logits, idx))
    rate_bits = (N * lse - sum_plog) / jnp.log(2.0)
    return (embed_hat, idx, rate_bits, vq_loss)


# speedup vs baseline: 1.1841x; 1.1841x over previous
import functools

import jax
import jax.numpy as jnp
from jax import lax
from jax.experimental import pallas as pl
from jax.experimental.pallas import tpu as pltpu
from jax.experimental.pallas import tpu_sc as plsc

H, D, K = 4096, 256, 8192
BETA = 0.25
N = 2 * 2048
BM = 256
NBLK = N // BM
KC = 2048
NKC = K // KC


def _vq_argmin_kernel(emb_ref, wpre_ref, bpre_ref, cb_ref, e2_ref,
                      z_ref, idx_ref, part_ref):
    z = jax.lax.dot_general(
        emb_ref[...], wpre_ref[...],
        dimension_numbers=(((1,), (1,)), ((), ())),
        preferred_element_type=jnp.float32)
    z = z + bpre_ref[...]
    z_ref[...] = z
    x2 = jnp.sum(z ** 2, axis=1, keepdims=True)
    z2 = z + z

    fiota = jax.lax.broadcasted_iota(jnp.int32, (BM, KC), 1).astype(jnp.float32)
    mvec = None
    ckvec = None
    for c in range(NKC):
        xe2 = jax.lax.dot_general(
            z2, cb_ref[pl.ds(c * KC, KC), :],
            dimension_numbers=(((1,), (1,)), ((), ())),
            preferred_element_type=jnp.float32)
        dist = (x2 + e2_ref[:, pl.ds(c * KC, KC)]) - xe2
        if c == 0:
            mvec = dist
            ckvec = jnp.zeros((BM, KC), jnp.float32)
        else:
            upd = dist < mvec
            mvec = jnp.where(upd, dist, mvec)
            ckvec = jnp.where(upd, jnp.float32(c * KC), ckvec)

    m = jnp.min(mvec, axis=1, keepdims=True)
    idxf = jnp.min(jnp.where(mvec == m, ckvec + fiota, jnp.float32(K)),
                   axis=1, keepdims=True)
    idx_ref[0, :, :] = idxf.astype(jnp.int32)

    sum_sq = jnp.sum(m)
    lane = jax.lax.broadcasted_iota(jnp.int32, (1, 128), 1)
    part_ref[0, ...] = jnp.where(lane == 0, sum_sq, 0.0)


def _post_kernel(z_ref, xq_ref, wpost_ref, bpost_ref, out_ref):
    z = z_ref[...]
    x_q = xq_ref[...]
    x_q_st = z + (x_q - z)
    out = jax.lax.dot_general(
        x_q_st, wpost_ref[...],
        dimension_numbers=(((1,), (1,)), ((), ())),
        preferred_element_type=jnp.float32)
    out_ref[...] = out + bpost_ref[...]


_SC_WORKERS = 32
_BPW = N // _SC_WORKERS


@functools.partial(
    pl.kernel,
    mesh=plsc.VectorSubcoreMesh(core_axis_name="c", subcore_axis_name="s"),
    out_type=jax.ShapeDtypeStruct((N, D), jnp.float32),
    scratch_types=[
        pltpu.VMEM((_BPW,), jnp.int32),
        pltpu.VMEM((_BPW, D), jnp.float32),
        pltpu.SemaphoreType.DMA,
    ],
)
def _sc_gather_kernel(table_hbm, idx_hbm, out_hbm, idx_v, rows_v, sem):
    wid = lax.axis_index("s") * 2 + lax.axis_index("c")
    base = wid * _BPW
    pltpu.sync_copy(idx_hbm.at[pl.ds(base, _BPW)], idx_v)
    pltpu.async_copy(table_hbm.at[idx_v], rows_v, sem).wait()
    pltpu.sync_copy(rows_v, out_hbm.at[pl.ds(base, _BPW)])


def kernel(embed, W_pre, b_pre, codebook, W_post, b_post, prior_logits):
    emb2d = embed.reshape(N, H)
    e2 = jnp.sum(codebook ** 2, axis=1)[None, :]

    z, idx3, parts = pl.pallas_call(
        _vq_argmin_kernel,
        grid=(NBLK,),
        compiler_params=pltpu.CompilerParams(
            dimension_semantics=("parallel",)),
        in_specs=[
            pl.BlockSpec((BM, H), lambda i: (i, 0)),
            pl.BlockSpec((D, H), lambda i: (0, 0)),
            pl.BlockSpec((1, D), lambda i: (0, 0)),
            pl.BlockSpec((K, D), lambda i: (0, 0)),
            pl.BlockSpec((1, K), lambda i: (0, 0)),
        ],
        out_specs=[
            pl.BlockSpec((BM, D), lambda i: (i, 0)),
            pl.BlockSpec((1, BM, 1), lambda i: (i, 0, 0)),
            pl.BlockSpec((1, 1, 128), lambda i: (i, 0, 0)),
        ],
        out_shape=[
            jax.ShapeDtypeStruct((N, D), jnp.float32),
            jax.ShapeDtypeStruct((NBLK, BM, 1), jnp.int32),
            jax.ShapeDtypeStruct((NBLK, 1, 128), jnp.float32),
        ],
    )(emb2d, W_pre, b_pre.reshape(1, D), codebook, e2)
    idx = idx3.reshape(N)

    x_q = _sc_gather_kernel(codebook, idx)

    embed_hat2d = pl.pallas_call(
        _post_kernel,
        grid=(NBLK,),
        compiler_params=pltpu.CompilerParams(
            dimension_semantics=("parallel",)),
        in_specs=[
            pl.BlockSpec((BM, D), lambda i: (i, 0)),
            pl.BlockSpec((BM, D), lambda i: (i, 0)),
            pl.BlockSpec((H, D), lambda i: (0, 0)),
            pl.BlockSpec((1, H), lambda i: (0, 0)),
        ],
        out_specs=pl.BlockSpec((BM, H), lambda i: (i, 0)),
        out_shape=jax.ShapeDtypeStruct((N, H), jnp.float32),
    )(z, x_q, W_post, b_post.reshape(1, H))

    embed_hat = embed_hat2d.reshape(embed.shape)
    mean_sq = jnp.sum(parts[:, 0, 0]) / (N * D)
    vq_loss = mean_sq + BETA * mean_sq
    lse = jax.nn.logsumexp(prior_logits)
    sum_plog = jnp.sum(jnp.take(prior_logits, idx))
    rate_bits = (N * lse - sum_plog) / jnp.log(2.0)
    return (embed_hat, idx, rate_bits, vq_loss)


# speedup vs baseline: 1.2107x; 1.0225x over previous
import functools

import jax
import jax.numpy as jnp
from jax import lax
from jax.experimental import pallas as pl
from jax.experimental.pallas import tpu as pltpu
from jax.experimental.pallas import tpu_sc as plsc

H, D, K = 4096, 256, 8192
BETA = 0.25
N = 2 * 2048
BM = 256
NBLK = N // BM
KC = 2048
NKC = K // KC


def _vq_argmin_kernel(emb_ref, wpre_ref, bpre_ref, cb_ref,
                      z_ref, idx_ref, part_ref, e2_ref):
    @pl.when(pl.program_id(0) == 0)
    def _():
        cb = cb_ref[...]
        e2_ref[...] = jax.lax.dot_general(
            jnp.ones((1, D), jnp.float32), cb * cb,
            dimension_numbers=(((1,), (1,)), ((), ())),
            preferred_element_type=jnp.float32)
    z = jax.lax.dot_general(
        emb_ref[...], wpre_ref[...],
        dimension_numbers=(((1,), (1,)), ((), ())),
        preferred_element_type=jnp.float32)
    z = z + bpre_ref[...]
    z_ref[...] = z
    x2 = jnp.sum(z ** 2, axis=1, keepdims=True)
    z2 = z + z

    fiota = jax.lax.broadcasted_iota(jnp.int32, (BM, KC), 1).astype(jnp.float32)
    mvec = None
    ckvec = None
    for c in range(NKC):
        xe2 = jax.lax.dot_general(
            z2, cb_ref[pl.ds(c * KC, KC), :],
            dimension_numbers=(((1,), (1,)), ((), ())),
            preferred_element_type=jnp.float32)
        dist = (x2 + e2_ref[:, pl.ds(c * KC, KC)]) - xe2
        if c == 0:
            mvec = dist
            ckvec = jnp.zeros((BM, KC), jnp.float32)
        else:
            upd = dist < mvec
            mvec = jnp.where(upd, dist, mvec)
            ckvec = jnp.where(upd, jnp.float32(c * KC), ckvec)

    m = jnp.min(mvec, axis=1, keepdims=True)
    idxf = jnp.min(jnp.where(mvec == m, ckvec + fiota, jnp.float32(K)),
                   axis=1, keepdims=True)
    idx_ref[0, :, :] = idxf.astype(jnp.int32)

    sum_sq = jnp.sum(m)
    lane = jax.lax.broadcasted_iota(jnp.int32, (1, 128), 1)
    part_ref[0, ...] = jnp.where(lane == 0, sum_sq, 0.0)


def _post_kernel(z_ref, xq_ref, wpost_ref, bpost_ref, out_ref):
    z = z_ref[...]
    x_q = xq_ref[...]
    x_q_st = z + (x_q - z)
    out = jax.lax.dot_general(
        x_q_st, wpost_ref[...],
        dimension_numbers=(((1,), (1,)), ((), ())),
        preferred_element_type=jnp.float32)
    out_ref[...] = out + bpost_ref[...]


_SC_WORKERS = 32
_BPW = N // _SC_WORKERS


@functools.partial(
    pl.kernel,
    mesh=plsc.VectorSubcoreMesh(core_axis_name="c", subcore_axis_name="s"),
    out_type=jax.ShapeDtypeStruct((N, D), jnp.float32),
    scratch_types=[
        pltpu.VMEM((_BPW,), jnp.int32),
        pltpu.VMEM((_BPW, D), jnp.float32),
        pltpu.SemaphoreType.DMA,
    ],
)
def _sc_gather_kernel(table_hbm, idx_hbm, out_hbm, idx_v, rows_v, sem):
    wid = lax.axis_index("s") * 2 + lax.axis_index("c")
    base = wid * _BPW
    pltpu.sync_copy(idx_hbm.at[pl.ds(base, _BPW)], idx_v)
    pltpu.async_copy(table_hbm.at[idx_v], rows_v, sem).wait()
    pltpu.sync_copy(rows_v, out_hbm.at[pl.ds(base, _BPW)])


def kernel(embed, W_pre, b_pre, codebook, W_post, b_post, prior_logits):
    emb2d = embed.reshape(N, H)

    z, idx3, parts = pl.pallas_call(
        _vq_argmin_kernel,
        grid=(NBLK,),
        compiler_params=pltpu.CompilerParams(
            dimension_semantics=("arbitrary",)),
        in_specs=[
            pl.BlockSpec((BM, H), lambda i: (i, 0)),
            pl.BlockSpec((D, H), lambda i: (0, 0)),
            pl.BlockSpec((1, D), lambda i: (0, 0)),
            pl.BlockSpec((K, D), lambda i: (0, 0)),
        ],
        out_specs=[
            pl.BlockSpec((BM, D), lambda i: (i, 0)),
            pl.BlockSpec((1, BM, 1), lambda i: (i, 0, 0)),
            pl.BlockSpec((1, 1, 128), lambda i: (i, 0, 0)),
        ],
        out_shape=[
            jax.ShapeDtypeStruct((N, D), jnp.float32),
            jax.ShapeDtypeStruct((NBLK, BM, 1), jnp.int32),
            jax.ShapeDtypeStruct((NBLK, 1, 128), jnp.float32),
        ],
        scratch_shapes=[pltpu.VMEM((1, K), jnp.float32)],
    )(emb2d, W_pre, b_pre.reshape(1, D), codebook)
    idx = idx3.reshape(N)

    x_q = _sc_gather_kernel(codebook, idx)

    embed_hat2d = pl.pallas_call(
        _post_kernel,
        grid=(NBLK,),
        compiler_params=pltpu.CompilerParams(
            dimension_semantics=("parallel",)),
        in_specs=[
            pl.BlockSpec((BM, D), lambda i: (i, 0)),
            pl.BlockSpec((BM, D), lambda i: (i, 0)),
            pl.BlockSpec((H, D), lambda i: (0, 0)),
            pl.BlockSpec((1, H), lambda i: (0, 0)),
        ],
        out_specs=pl.BlockSpec((BM, H), lambda i: (i, 0)),
        out_shape=jax.ShapeDtypeStruct((N, H), jnp.float32),
    )(z, x_q, W_post, b_post.reshape(1, H))

    embed_hat = embed_hat2d.reshape(embed.shape)
    mean_sq = jnp.sum(parts[:, 0, 0]) / (N * D)
    vq_loss = mean_sq + BETA * mean_sq
    lse = jax.nn.logsumexp(prior_logits)
    sum_plog = jnp.sum(jnp.take(prior_logits, idx))
    rate_bits = (N * lse - sum_plog) / jnp.log(2.0)
    return (embed_hat, idx, rate_bits, vq_loss)


# speedup vs baseline: 1.2108x; 1.0001x over previous
import functools

import jax
import jax.numpy as jnp
from jax import lax
from jax.experimental import pallas as pl
from jax.experimental.pallas import tpu as pltpu
from jax.experimental.pallas import tpu_sc as plsc

H, D, K = 4096, 256, 8192
BETA = 0.25
N = 2 * 2048
BM = 256
NBLK = N // BM
KC = 1024
NKC = K // KC


def _vq_argmin_kernel(emb_ref, wpre_ref, bpre_ref, cb_ref,
                      z_ref, idx_ref, part_ref, e2_ref):
    @pl.when(pl.program_id(0) == 0)
    def _():
        cb = cb_ref[...]
        e2_ref[...] = jax.lax.dot_general(
            jnp.ones((1, D), jnp.float32), cb * cb,
            dimension_numbers=(((1,), (1,)), ((), ())),
            preferred_element_type=jnp.float32)
    z = jax.lax.dot_general(
        emb_ref[...], wpre_ref[...],
        dimension_numbers=(((1,), (1,)), ((), ())),
        preferred_element_type=jnp.float32)
    z = z + bpre_ref[...]
    z_ref[...] = z
    x2 = jnp.sum(z ** 2, axis=1, keepdims=True)
    z2 = z + z

    fiota = jax.lax.broadcasted_iota(jnp.int32, (BM, KC), 1).astype(jnp.float32)
    mvec = None
    ckvec = None
    for c in range(NKC):
        xe2 = jax.lax.dot_general(
            z2, cb_ref[pl.ds(c * KC, KC), :],
            dimension_numbers=(((1,), (1,)), ((), ())),
            preferred_element_type=jnp.float32)
        dist = (x2 + e2_ref[:, pl.ds(c * KC, KC)]) - xe2
        if c == 0:
            mvec = dist
            ckvec = jnp.zeros((BM, KC), jnp.float32)
        else:
            upd = dist < mvec
            mvec = jnp.where(upd, dist, mvec)
            ckvec = jnp.where(upd, jnp.float32(c * KC), ckvec)

    m = jnp.min(mvec, axis=1, keepdims=True)
    idxf = jnp.min(jnp.where(mvec == m, ckvec + fiota, jnp.float32(K)),
                   axis=1, keepdims=True)
    idx_ref[0, :, :] = idxf.astype(jnp.int32)

    sum_sq = jnp.sum(m)
    lane = jax.lax.broadcasted_iota(jnp.int32, (1, 128), 1)
    part_ref[0, ...] = jnp.where(lane == 0, sum_sq, 0.0)


def _post_kernel(z_ref, xq_ref, wpost_ref, bpost_ref, out_ref):
    z = z_ref[...]
    x_q = xq_ref[...]
    x_q_st = z + (x_q - z)
    out = jax.lax.dot_general(
        x_q_st, wpost_ref[...],
        dimension_numbers=(((1,), (1,)), ((), ())),
        preferred_element_type=jnp.float32)
    out_ref[...] = out + bpost_ref[...]


_SC_WORKERS = 32
_BPW = N // _SC_WORKERS


@functools.partial(
    pl.kernel,
    mesh=plsc.VectorSubcoreMesh(core_axis_name="c", subcore_axis_name="s"),
    out_type=jax.ShapeDtypeStruct((N, D), jnp.float32),
    scratch_types=[
        pltpu.VMEM((_BPW,), jnp.int32),
        pltpu.VMEM((_BPW, D), jnp.float32),
        pltpu.SemaphoreType.DMA,
    ],
)
def _sc_gather_kernel(table_hbm, idx_hbm, out_hbm, idx_v, rows_v, sem):
    wid = lax.axis_index("s") * 2 + lax.axis_index("c")
    base = wid * _BPW
    pltpu.sync_copy(idx_hbm.at[pl.ds(base, _BPW)], idx_v)
    pltpu.async_copy(table_hbm.at[idx_v], rows_v, sem).wait()
    pltpu.sync_copy(rows_v, out_hbm.at[pl.ds(base, _BPW)])


def kernel(embed, W_pre, b_pre, codebook, W_post, b_post, prior_logits):
    emb2d = embed.reshape(N, H)

    z, idx3, parts = pl.pallas_call(
        _vq_argmin_kernel,
        grid=(NBLK,),
        compiler_params=pltpu.CompilerParams(
            dimension_semantics=("arbitrary",)),
        in_specs=[
            pl.BlockSpec((BM, H), lambda i: (i, 0)),
            pl.BlockSpec((D, H), lambda i: (0, 0)),
            pl.BlockSpec((1, D), lambda i: (0, 0)),
            pl.BlockSpec((K, D), lambda i: (0, 0)),
        ],
        out_specs=[
            pl.BlockSpec((BM, D), lambda i: (i, 0)),
            pl.BlockSpec((1, BM, 1), lambda i: (i, 0, 0)),
            pl.BlockSpec((1, 1, 128), lambda i: (i, 0, 0)),
        ],
        out_shape=[
            jax.ShapeDtypeStruct((N, D), jnp.float32),
            jax.ShapeDtypeStruct((NBLK, BM, 1), jnp.int32),
            jax.ShapeDtypeStruct((NBLK, 1, 128), jnp.float32),
        ],
        scratch_shapes=[pltpu.VMEM((1, K), jnp.float32)],
    )(emb2d, W_pre, b_pre.reshape(1, D), codebook)
    idx = idx3.reshape(N)

    x_q = _sc_gather_kernel(codebook, idx)

    embed_hat2d = pl.pallas_call(
        _post_kernel,
        grid=(NBLK,),
        compiler_params=pltpu.CompilerParams(
            dimension_semantics=("parallel",)),
        in_specs=[
            pl.BlockSpec((BM, D), lambda i: (i, 0)),
            pl.BlockSpec((BM, D), lambda i: (i, 0)),
            pl.BlockSpec((H, D), lambda i: (0, 0)),
            pl.BlockSpec((1, H), lambda i: (0, 0)),
        ],
        out_specs=pl.BlockSpec((BM, H), lambda i: (i, 0)),
        out_shape=jax.ShapeDtypeStruct((N, H), jnp.float32),
    )(z, x_q, W_post, b_post.reshape(1, H))

    embed_hat = embed_hat2d.reshape(embed.shape)
    mean_sq = jnp.sum(parts[:, 0, 0]) / (N * D)
    vq_loss = mean_sq + BETA * mean_sq
    lse = jax.nn.logsumexp(prior_logits)
    sum_plog = jnp.sum(jnp.take(prior_logits, idx))
    rate_bits = (N * lse - sum_plog) / jnp.log(2.0)
    return (embed_hat, idx, rate_bits, vq_loss)
